# baseline (speedup 1.0000x reference)
import functools

import jax
import jax.numpy as jnp
from jax import lax
from jax.experimental import pallas as pl
from jax.experimental.pallas import tpu as pltpu
from jax.experimental.pallas import tpu_sc as plsc

N = 10000
E = 320000
D = 128
DS = 64
ALPHA = 0.2

NC = 2
NS = 16
EPT = E // NS
K = 80
NB = EPT // K
NP_ = 10240
RPT = NP_ // NS
ZR = 128
G = K // 16

_TC_ROWS = 1000
_GRID = N // _TC_ROWS


def _prep_body(x_ref, w_ref, a1_ref, a2_ref, hs_ref, f1_ref, f2_ref):
    h = jnp.dot(x_ref[...], w_ref[...], preferred_element_type=jnp.float32)
    hs_ref[0, :, :] = h[:, :DS]
    hs_ref[1, :, :] = h[:, DS:]
    f1_ref[...] = jnp.dot(h, a1_ref[...], preferred_element_type=jnp.float32)
    f2_ref[...] = jnp.dot(h, a2_ref[...], preferred_element_type=jnp.float32)


def _prep(x, W, a1, a2):
    return pl.pallas_call(
        _prep_body,
        grid=(_GRID,),
        in_specs=[
            pl.BlockSpec((_TC_ROWS, D), lambda i: (i, 0)),
            pl.BlockSpec((D, D), lambda i: (0, 0)),
            pl.BlockSpec((D, 1), lambda i: (0, 0)),
            pl.BlockSpec((D, 1), lambda i: (0, 0)),
        ],
        out_specs=[
            pl.BlockSpec((2, _TC_ROWS, DS), lambda i: (0, i, 0)),
            pl.BlockSpec((_TC_ROWS, 1), lambda i: (i, 0)),
            pl.BlockSpec((_TC_ROWS, 1), lambda i: (i, 0)),
        ],
        out_shape=[
            jax.ShapeDtypeStruct((2, N, DS), jnp.float32),
            jax.ShapeDtypeStruct((N, 1), jnp.float32),
            jax.ShapeDtypeStruct((N, 1), jnp.float32),
        ],
    )(x, W, a1, a2)


def _finish_body(p0_ref, p1_ref, rs0_ref, rs1_ref, out_ref):
    rs = rs0_ref[...] + rs1_ref[...] + 1e-15
    r = jnp.concatenate([p0_ref[...], p1_ref[...]], axis=1) / rs
    out_ref[...] = jnp.where(r > 0, r, jnp.exp(jnp.minimum(r, 0.0)) - 1.0)


def _finish(hp2, rs):
    grid = NP_ // RPT
    return pl.pallas_call(
        _finish_body,
        grid=(grid,),
        in_specs=[
            pl.BlockSpec((RPT, DS), lambda i: (i, 0)),
            pl.BlockSpec((RPT, DS), lambda i: (i + NP_ // RPT, 0)),
            pl.BlockSpec((RPT, 1), lambda i: (i, 0)),
            pl.BlockSpec((RPT, 1), lambda i: (i + NP_ // RPT, 0)),
        ],
        out_specs=pl.BlockSpec((RPT, D), lambda i: (i, 0)),
        out_shape=jax.ShapeDtypeStruct((NP_, D), jnp.float32),
    )(hp2, hp2, rs, rs)


def _sc_edge_factory():
    mesh = plsc.VectorSubcoreMesh(core_axis_name="c", subcore_axis_name="s")

    @functools.partial(
        pl.kernel,
        out_type=[
            jax.ShapeDtypeStruct((2 * NP_, DS), jnp.float32),
            jax.ShapeDtypeStruct((2, NP_), jnp.float32),
        ],
        mesh=mesh,
        compiler_params=pltpu.CompilerParams(needs_layout_passes=False,
                                             use_tc_tiling_on_sc=False),
        scratch_types=[
            pltpu.VMEM((NB, K), jnp.int32),
            pltpu.VMEM((NB, K), jnp.int32),
            pltpu.VMEM((N,), jnp.float32),
            pltpu.VMEM((N,), jnp.float32),
            pltpu.VMEM((K,), jnp.float32),
            pltpu.VMEM((ZR, DS), jnp.float32),
            pltpu.VMEM((RPT,), jnp.float32),
            pltpu.VMEM((2, K, DS), jnp.float32),
            pltpu.VMEM_SHARED((NP_, DS), jnp.float32),
            pltpu.VMEM_SHARED((NP_,), jnp.float32),
            pltpu.SemaphoreType.DMA((2,)),
        ],
    )
    def sc_edge(src_hbm, dst_hbm, f1_hbm, f2_hbm, h_hbm, hp_out, rs_out,
                src_v, dst_v, f1_v, f2_v, ev_v, zb_v, zr_v, rows2, acc,
                rs_acc, gsem2):
        cid = lax.axis_index("c")
        sid = lax.axis_index("s")
        hs = h_hbm.at[cid]

        pltpu.sync_copy(src_hbm.at[sid], src_v)
        pltpu.sync_copy(dst_hbm.at[sid], dst_v)
        pltpu.sync_copy(f1_hbm, f1_v)
        pltpu.sync_copy(f2_hbm, f2_v)

        def _zero_row(r):
            for c in range(DS // 16):
                zb_v[r, pl.ds(c * 16, 16)] = jnp.zeros((16,), jnp.float32)
        pl.loop(0, ZR)(_zero_row)

        def _zero_rs(r):
            zr_v[pl.ds(r * 16, 16)] = jnp.zeros((16,), jnp.float32)
        pl.loop(0, RPT // 16)(_zero_rs)

        for z in range(RPT // ZR):
            pltpu.sync_copy(zb_v, acc.at[pl.ds(sid * RPT + z * ZR, ZR)])
        pltpu.sync_copy(zr_v, rs_acc.at[pl.ds(sid * RPT, RPT)])
        plsc.subcore_barrier()

        def _ev_batch(b):
            for g in range(G):
                s16 = src_v[b, pl.ds(g * 16, 16)]
                d16 = dst_v[b, pl.ds(g * 16, 16)]
                v = (plsc.load_gather(f1_v, [s16])
                     + plsc.load_gather(f2_v, [d16]))
                v = jnp.maximum(v, ALPHA * v)
                ev_v[pl.ds(g * 16, 16)] = jnp.exp(v)

        def _start_g(b, p):
            pltpu.async_copy(hs.at[dst_v.at[b]], rows2.at[p], gsem2.at[p])

        def _slot(b):
            p = lax.rem(b, 2)
            _ev_batch(b)
            pltpu.make_async_copy(hs.at[dst_v.at[b]], rows2.at[p],
                                  gsem2.at[p]).wait()

            def _scale_g(g):
                ev16 = ev_v[pl.ds(g * 16, 16)]
                for e0 in range(16):
                    sc = ev16[e0]
                    r = g * 16 + e0
                    for c in range(DS // 16):
                        rows2[p, r, pl.ds(c * 16, 16)] = (
                            rows2[p, r, pl.ds(c * 16, 16)] * sc)
            pl.loop(0, G)(_scale_g)

            @pl.when(b + 1 < NB)
            def _():
                _start_g(b + 1, 1 - p)

            pltpu.sync_copy(rows2.at[p], acc.at[src_v.at[b]], add=True)

            @pl.when(p == cid)
            def _():
                pltpu.sync_copy(ev_v, rs_acc.at[src_v.at[b]], add=True)

        _start_g(0, 0)
        pl.loop(0, NB)(_slot)
        plsc.subcore_barrier()

        pltpu.sync_copy(acc.at[pl.ds(sid * RPT, RPT)],
                        hp_out.at[pl.ds(cid * NP_ + sid * RPT, RPT)])

        pltpu.sync_copy(rs_acc.at[pl.ds(sid * RPT, RPT)],
                        rs_out.at[cid].at[pl.ds(sid * RPT, RPT)])

    return sc_edge


_sc_edge = _sc_edge_factory()


def kernel(inputs, edge_index, W, a):
    a1 = a[0, :D].reshape(D, 1)
    a2 = a[0, D:].reshape(D, 1)
    h_sp, f1, f2 = _prep(inputs, W, a1, a2)
    src3d = edge_index[0].reshape(NS, NB, K)
    dst3d = edge_index[1].reshape(NS, NB, K)
    hp2, rs = _sc_edge(src3d, dst3d, f1.reshape(N), f2.reshape(N), h_sp)
    return _finish(hp2, rs.reshape(2 * NP_, 1))[:N]


# speedup vs baseline: 1.3667x; 1.3667x over previous
import functools

import jax
import jax.numpy as jnp
from jax import lax
from jax.experimental import pallas as pl
from jax.experimental.pallas import tpu as pltpu
from jax.experimental.pallas import tpu_sc as plsc

N = 10000
E = 320000
D = 128
DS = 64
ALPHA = 0.2

NC = 2
NS = 16
EPT = E // NS
K = 80
NB = EPT // K
NP_ = 10240
RPT = NP_ // NS
ZR = 128
G = K // 16

_TC_ROWS = 1000
_GRID = N // _TC_ROWS


def _prep_body(x_ref, w_ref, a1_ref, a2_ref, hs_ref, f1_ref, f2_ref):
    h = jnp.dot(x_ref[...], w_ref[...], preferred_element_type=jnp.float32)
    hs_ref[0, :, :] = h[:, :DS]
    hs_ref[1, :, :] = h[:, DS:]
    f1_ref[...] = jnp.dot(h, a1_ref[...], preferred_element_type=jnp.float32)
    f2_ref[...] = jnp.dot(h, a2_ref[...], preferred_element_type=jnp.float32)


def _prep(x, W, a1, a2):
    return pl.pallas_call(
        _prep_body,
        grid=(_GRID,),
        in_specs=[
            pl.BlockSpec((_TC_ROWS, D), lambda i: (i, 0)),
            pl.BlockSpec((D, D), lambda i: (0, 0)),
            pl.BlockSpec((D, 1), lambda i: (0, 0)),
            pl.BlockSpec((D, 1), lambda i: (0, 0)),
        ],
        out_specs=[
            pl.BlockSpec((2, _TC_ROWS, DS), lambda i: (0, i, 0)),
            pl.BlockSpec((_TC_ROWS, 1), lambda i: (i, 0)),
            pl.BlockSpec((_TC_ROWS, 1), lambda i: (i, 0)),
        ],
        out_shape=[
            jax.ShapeDtypeStruct((2, N, DS), jnp.float32),
            jax.ShapeDtypeStruct((N, 1), jnp.float32),
            jax.ShapeDtypeStruct((N, 1), jnp.float32),
        ],
    )(x, W, a1, a2)


def _finish_body(p0_ref, p1_ref, rs0_ref, rs1_ref, out_ref):
    rs = rs0_ref[...] + rs1_ref[...] + 1e-15
    r = jnp.concatenate([p0_ref[...], p1_ref[...]], axis=1) / rs
    out_ref[...] = jnp.where(r > 0, r, jnp.exp(jnp.minimum(r, 0.0)) - 1.0)


def _finish(hp2, rs):
    grid = NP_ // RPT
    return pl.pallas_call(
        _finish_body,
        grid=(grid,),
        in_specs=[
            pl.BlockSpec((RPT, DS), lambda i: (i, 0)),
            pl.BlockSpec((RPT, DS), lambda i: (i + NP_ // RPT, 0)),
            pl.BlockSpec((RPT, 1), lambda i: (i, 0)),
            pl.BlockSpec((RPT, 1), lambda i: (i + NP_ // RPT, 0)),
        ],
        out_specs=pl.BlockSpec((RPT, D), lambda i: (i, 0)),
        out_shape=jax.ShapeDtypeStruct((NP_, D), jnp.float32),
    )(hp2, hp2, rs, rs)


def _sc_edge_factory():
    mesh = plsc.VectorSubcoreMesh(core_axis_name="c", subcore_axis_name="s")

    @functools.partial(
        pl.kernel,
        out_type=[
            jax.ShapeDtypeStruct((2 * NP_, DS), jnp.float32),
            jax.ShapeDtypeStruct((2, NP_), jnp.float32),
        ],
        mesh=mesh,
        compiler_params=pltpu.CompilerParams(needs_layout_passes=False,
                                             use_tc_tiling_on_sc=False),
        scratch_types=[
            pltpu.VMEM((NB, K), jnp.int32),
            pltpu.VMEM((NB, K), jnp.int32),
            pltpu.VMEM((N,), jnp.float32),
            pltpu.VMEM((N,), jnp.float32),
            pltpu.VMEM((2, K), jnp.float32),
            pltpu.VMEM((ZR, DS), jnp.float32),
            pltpu.VMEM((RPT,), jnp.float32),
            pltpu.VMEM((4, K, DS), jnp.float32),
            pltpu.VMEM_SHARED((NP_, DS), jnp.float32),
            pltpu.VMEM_SHARED((NP_,), jnp.float32),
            pltpu.SemaphoreType.DMA((4,)),
            pltpu.SemaphoreType.DMA((4,)),
            pltpu.SemaphoreType.DMA((2,)),
        ],
    )
    def sc_edge(src_hbm, dst_hbm, f1_hbm, f2_hbm, h_hbm, hp_out, rs_out,
                src_v, dst_v, f1_v, f2_v, ev2, zb_v, zr_v, rows4, acc,
                rs_acc, gsem4, ssem4, rsem2):
        cid = lax.axis_index("c")
        sid = lax.axis_index("s")
        hs = h_hbm.at[cid]

        pltpu.sync_copy(src_hbm.at[sid], src_v)
        pltpu.sync_copy(dst_hbm.at[sid], dst_v)
        pltpu.sync_copy(f1_hbm, f1_v)
        pltpu.sync_copy(f2_hbm, f2_v)

        def _zero_row(r):
            for c in range(DS // 16):
                zb_v[r, pl.ds(c * 16, 16)] = jnp.zeros((16,), jnp.float32)
        pl.loop(0, ZR)(_zero_row)

        def _zero_rs(r):
            zr_v[pl.ds(r * 16, 16)] = jnp.zeros((16,), jnp.float32)
        pl.loop(0, RPT // 16)(_zero_rs)

        for z in range(RPT // ZR):
            pltpu.sync_copy(zb_v, acc.at[pl.ds(sid * RPT + z * ZR, ZR)])
        pltpu.sync_copy(zr_v, rs_acc.at[pl.ds(sid * RPT, RPT)])
        plsc.subcore_barrier()

        def _ev_batch(b, pe):
            for g in range(G):
                s16 = src_v[b, pl.ds(g * 16, 16)]
                d16 = dst_v[b, pl.ds(g * 16, 16)]
                v = (plsc.load_gather(f1_v, [s16])
                     + plsc.load_gather(f2_v, [d16]))
                v = jnp.maximum(v, ALPHA * v)
                ev2[pe, pl.ds(g * 16, 16)] = jnp.exp(v)

        def _start_g(b, p):
            pltpu.async_copy(hs.at[dst_v.at[b]], rows4.at[p], gsem4.at[p])

        def _wait_g(b, p):
            pltpu.make_async_copy(hs.at[dst_v.at[b]], rows4.at[p],
                                  gsem4.at[p]).wait()

        def _start_s(b, p):
            pltpu.async_copy(rows4.at[p], acc.at[src_v.at[b]], ssem4.at[p],
                             add=True)

        def _wait_s(b, p):
            pltpu.make_async_copy(rows4.at[p], acc.at[src_v.at[b]],
                                  ssem4.at[p]).wait()

        def _start_rs(b, pe):
            pltpu.async_copy(ev2.at[pe], rs_acc.at[src_v.at[b]],
                             rsem2.at[pe], add=True)

        def _wait_rs(b, pe):
            pltpu.make_async_copy(ev2.at[pe], rs_acc.at[src_v.at[b]],
                                  rsem2.at[pe]).wait()

        def _slot(b):
            p = lax.rem(b, 4)
            pe = lax.rem(b, 2)

            @pl.when(jnp.logical_and(b >= 2, pe == cid))
            def _():
                _wait_rs(b - 2, pe)
            _ev_batch(b, pe)

            _wait_g(b, p)

            def _scale_g(g):
                ev16 = ev2[pe, pl.ds(g * 16, 16)]
                for e0 in range(16):
                    sc = ev16[e0]
                    r = g * 16 + e0
                    for c in range(DS // 16):
                        rows4[p, r, pl.ds(c * 16, 16)] = (
                            rows4[p, r, pl.ds(c * 16, 16)] * sc)
            pl.loop(0, G)(_scale_g)

            _start_s(b, p)

            @pl.when(pe == cid)
            def _():
                _start_rs(b, pe)

            q = lax.rem(b + 2, 4)

            @pl.when(b >= 2)
            def _():
                _wait_s(b - 2, q)

            @pl.when(b + 2 < NB)
            def _():
                _start_g(b + 2, q)

        _start_g(0, 0)
        _start_g(1, 1)
        pl.loop(0, NB)(_slot)
        _wait_s(NB - 2, lax.rem(NB - 2, 4))
        _wait_s(NB - 1, lax.rem(NB - 1, 4))

        @pl.when(cid == 0)
        def _():
            _wait_rs(NB - 2, 0)

        @pl.when(cid == 1)
        def _():
            _wait_rs(NB - 1, 1)
        plsc.subcore_barrier()

        pltpu.sync_copy(acc.at[pl.ds(sid * RPT, RPT)],
                        hp_out.at[pl.ds(cid * NP_ + sid * RPT, RPT)])

        pltpu.sync_copy(rs_acc.at[pl.ds(sid * RPT, RPT)],
                        rs_out.at[cid].at[pl.ds(sid * RPT, RPT)])

    return sc_edge


_sc_edge = _sc_edge_factory()


def kernel(inputs, edge_index, W, a):
    a1 = a[0, :D].reshape(D, 1)
    a2 = a[0, D:].reshape(D, 1)
    h_sp, f1, f2 = _prep(inputs, W, a1, a2)
    src3d = edge_index[0].reshape(NS, NB, K)
    dst3d = edge_index[1].reshape(NS, NB, K)
    hp2, rs = _sc_edge(src3d, dst3d, f1.reshape(N), f2.reshape(N), h_sp)
    return _finish(hp2, rs.reshape(2 * NP_, 1))[:N]
